# baseline (speedup 1.0000x reference)
import jax
import jax.numpy as jnp
from jax.experimental import pallas as pl
from jax.experimental.pallas import tpu as pltpu

_B = 2048
_D = 128
_DA = _D + 8
_NCLS = 256
_MARGIN = 1.0
_NQ = 8
_S = _B // _NQ


def _dist(lhs, rhs):
    return jax.lax.dot_general(lhs, rhs, (((1,), (1,)), ((), ())),
                               preferred_element_type=jnp.float32)


def _triplet_kernel(e_ref, labc_ref, labr_ref, sum_ref, cnt_ref,
                    lhs_ref, rhs_ref):
    labr = labr_ref[...]
    labc = labc_ref[...]
    inf = jnp.float32(jnp.inf)

    e = e_ref[...]
    sq = jnp.sum(e * e, axis=1, keepdims=True)
    one = jnp.ones((_B, 1), jnp.float32)
    zp = jnp.zeros((_B, _DA - _D - 2), jnp.float32)
    lhs_ref[:, 0:_D] = e
    lhs_ref[:, _D:_D + 1] = sq
    lhs_ref[:, _D + 1:_D + 2] = one
    lhs_ref[:, _D + 2:] = zp
    rhs_ref[:, 0:_D] = e * jnp.float32(-2.0)
    rhs_ref[:, _D:_D + 1] = one
    rhs_ref[:, _D + 1:_D + 2] = sq
    rhs_ref[:, _D + 2:] = zp
    cls = jax.lax.broadcasted_iota(jnp.int32, (_NCLS, 1), 0)
    ohc = jnp.where(cls == labr, 1.0, 0.0)
    ncls = jnp.sum(ohc, axis=1, keepdims=True)
    s1 = jnp.sum(ncls * ncls, keepdims=True)
    cnt = (0.5 * (s1 - jnp.float32(_B))).astype(jnp.int32)
    cnt_ref[...] = cnt

    colv = jax.lax.broadcasted_iota(jnp.int32, (1, _S), 1)
    rowv = jax.lax.broadcasted_iota(jnp.int32, (_S, 1), 0)

    sums = []
    cmins = {q: [] for q in range(_NQ)}
    fbrow = None
    for q in range(_NQ):
        c0 = q * _S
        dist_q = _dist(lhs_ref[c0:c0 + _S, :], rhs_ref[c0:, :])
        eq_q = labc[c0:c0 + _S] == labr[:, c0:]
        mh_q = jnp.where(eq_q, inf, dist_q)
        if q == 0:
            fbrow = dist_q[0:1, :]
        neg = jnp.min(mh_q, axis=1, keepdims=True)
        for t in range(q + 1, _NQ):
            off = t * _S - c0
            cmins[t].append(
                jnp.min(mh_q[:, off:off + _S], axis=0, keepdims=True).T)
        for cm in cmins[q]:
            neg = jnp.minimum(neg, cm)
        fbv = dist_q[:, 0:1] if q == 0 else fbrow[0:1, c0:c0 + _S].T
        neg = jnp.where(neg < inf, neg, fbv)
        x = jnp.maximum(dist_q - (neg - _MARGIN), 0.0)
        posd = jnp.logical_and(eq_q[:, 0:_S], colv > rowv)
        sums.append(jnp.sum(jnp.where(posd, x[:, 0:_S], 0.0),
                            keepdims=True))
        if q + 1 < _NQ:
            sums.append(jnp.sum(jnp.where(eq_q[:, _S:], x[:, _S:], 0.0),
                                keepdims=True))

    total = sums[0]
    for s in sums[1:]:
        total = total + s
    sum_ref[...] = total / cnt.astype(jnp.float32)


def kernel(embeddings, target):
    labc = target.reshape(_B, 1)
    labr = target.reshape(1, _B)
    out_sum, out_cnt = pl.pallas_call(
        _triplet_kernel,
        out_shape=[
            jax.ShapeDtypeStruct((1, 1), jnp.float32),
            jax.ShapeDtypeStruct((1, 1), jnp.int32),
        ],
        scratch_shapes=[
            pltpu.VMEM((_B, _DA), jnp.float32),
            pltpu.VMEM((_B, _DA), jnp.float32),
        ],
    )(embeddings, labc, labr)
    return (out_sum[0, 0], out_cnt[0, 0])


# speedup vs baseline: 1.1996x; 1.1996x over previous
import jax
import jax.numpy as jnp
from jax.experimental import pallas as pl
from jax.experimental.pallas import tpu as pltpu

_B = 2048
_D = 128
_DA = _D + 8
_NCLS = 256
_MARGIN = 1.0
_NQ = 4
_S = _B // _NQ


def _dist(lhs, rhs):
    return jax.lax.dot_general(lhs, rhs, (((1,), (1,)), ((), ())),
                               preferred_element_type=jnp.float32)


def _triplet_kernel(e_ref, labc_ref, labr_ref, sum_ref, cnt_ref,
                    lhs_ref, rhs_ref):
    labr = labr_ref[...]
    labc = labc_ref[...]
    inf = jnp.float32(jnp.inf)

    e = e_ref[...]
    sq = jnp.sum(e * e, axis=1, keepdims=True)
    one = jnp.ones((_B, 1), jnp.float32)
    zp = jnp.zeros((_B, _DA - _D - 2), jnp.float32)
    lhs_ref[:, 0:_D] = e
    lhs_ref[:, _D:_D + 1] = sq
    lhs_ref[:, _D + 1:_D + 2] = one
    lhs_ref[:, _D + 2:] = zp
    rhs_ref[:, 0:_D] = e * jnp.float32(-2.0)
    rhs_ref[:, _D:_D + 1] = one
    rhs_ref[:, _D + 1:_D + 2] = sq
    rhs_ref[:, _D + 2:] = zp
    cls = jax.lax.broadcasted_iota(jnp.int32, (_NCLS, 1), 0)
    ohc = jnp.where(cls == labr, 1.0, 0.0)
    ncls = jnp.sum(ohc, axis=1, keepdims=True)
    s1 = jnp.sum(ncls * ncls, keepdims=True)
    cnt = (0.5 * (s1 - jnp.float32(_B))).astype(jnp.int32)
    cnt_ref[...] = cnt

    colv = jax.lax.broadcasted_iota(jnp.int32, (1, _S), 1)
    rowv = jax.lax.broadcasted_iota(jnp.int32, (_S, 1), 0)

    sums = []
    cmins = {q: [] for q in range(_NQ)}
    fbrow = None
    for q in range(_NQ):
        c0 = q * _S
        dist_q = _dist(lhs_ref[c0:c0 + _S, :], rhs_ref[c0:, :])
        eq_q = labc[c0:c0 + _S] == labr[:, c0:]
        mh_q = jnp.where(eq_q, inf, dist_q)
        if q == 0:
            fbrow = dist_q[0:1, :]
        neg = jnp.min(mh_q, axis=1, keepdims=True)
        for t in range(q + 1, _NQ):
            off = t * _S - c0
            cmins[t].append(
                jnp.min(mh_q[:, off:off + _S], axis=0, keepdims=True).T)
        for cm in cmins[q]:
            neg = jnp.minimum(neg, cm)
        fbv = dist_q[:, 0:1] if q == 0 else fbrow[0:1, c0:c0 + _S].T
        neg = jnp.where(neg < inf, neg, fbv)
        x = jnp.maximum(dist_q - (neg - _MARGIN), 0.0)
        posd = jnp.logical_and(eq_q[:, 0:_S], colv > rowv)
        sums.append(jnp.sum(jnp.where(posd, x[:, 0:_S], 0.0),
                            keepdims=True))
        if q + 1 < _NQ:
            sums.append(jnp.sum(jnp.where(eq_q[:, _S:], x[:, _S:], 0.0),
                                keepdims=True))

    total = sums[0]
    for s in sums[1:]:
        total = total + s
    sum_ref[...] = total / cnt.astype(jnp.float32)


def kernel(embeddings, target):
    labc = target.reshape(_B, 1)
    labr = target.reshape(1, _B)
    out_sum, out_cnt = pl.pallas_call(
        _triplet_kernel,
        out_shape=[
            jax.ShapeDtypeStruct((1, 1), jnp.float32),
            jax.ShapeDtypeStruct((1, 1), jnp.int32),
        ],
        scratch_shapes=[
            pltpu.VMEM((_B, _DA), jnp.float32),
            pltpu.VMEM((_B, _DA), jnp.float32),
        ],
    )(embeddings, labc, labr)
    return (out_sum[0, 0], out_cnt[0, 0])


# speedup vs baseline: 1.2027x; 1.0026x over previous
import jax
import jax.numpy as jnp
from jax.experimental import pallas as pl
from jax.experimental.pallas import tpu as pltpu

_B = 2048
_D = 128
_DA = _D + 8
_NCLS = 256
_MARGIN = 1.0
_NQ = 4
_S = _B // _NQ


def _dist(lhs, rhs):
    return jax.lax.dot_general(lhs, rhs, (((1,), (1,)), ((), ())),
                               preferred_element_type=jnp.float32)


def _triplet_kernel(e_ref, labc_ref, labr_ref, sum_ref, cnt_ref,
                    lhs_ref, rhs_ref):
    labr = labr_ref[...]
    labc = labc_ref[...]
    inf = jnp.float32(jnp.inf)

    e = e_ref[...]
    sq = jnp.sum(e * e, axis=1, keepdims=True)
    one = jnp.ones((_B, 1), jnp.float32)
    zp = jnp.zeros((_B, _DA - _D - 2), jnp.float32)
    lhs_ref[:, 0:_D] = e
    lhs_ref[:, _D:_D + 1] = sq
    lhs_ref[:, _D + 1:_D + 2] = one
    lhs_ref[:, _D + 2:] = zp
    rhs_ref[:, 0:_D] = e * jnp.float32(-2.0)
    rhs_ref[:, _D:_D + 1] = one
    rhs_ref[:, _D + 1:_D + 2] = sq
    rhs_ref[:, _D + 2:] = zp
    cls = jax.lax.broadcasted_iota(jnp.int32, (_NCLS, 1), 0)
    ohc = jnp.where(cls == labr, 1.0, 0.0)
    ncls = jnp.sum(ohc, axis=1, keepdims=True)
    s1 = jnp.sum(ncls * ncls, keepdims=True)
    cnt = (0.5 * (s1 - jnp.float32(_B))).astype(jnp.int32)
    cnt_ref[...] = cnt

    colv = jax.lax.broadcasted_iota(jnp.int32, (1, _S), 1)
    rowv = jax.lax.broadcasted_iota(jnp.int32, (_S, 1), 0)

    sums = []
    cmins = {q: [] for q in range(_NQ)}
    fbrow = None
    dists = [_dist(lhs_ref[q * _S:(q + 1) * _S, :], rhs_ref[q * _S:, :])
             for q in range(_NQ)]
    for q in range(_NQ):
        c0 = q * _S
        dist_q = dists[q]
        eq_q = labc[c0:c0 + _S] == labr[:, c0:]
        mh_q = jnp.where(eq_q, inf, dist_q)
        if q == 0:
            fbrow = dist_q[0:1, :]
        neg = jnp.min(mh_q, axis=1, keepdims=True)
        for t in range(q + 1, _NQ):
            off = t * _S - c0
            cmins[t].append(
                jnp.min(mh_q[:, off:off + _S], axis=0, keepdims=True).T)
        for cm in cmins[q]:
            neg = jnp.minimum(neg, cm)
        fbv = dist_q[:, 0:1] if q == 0 else fbrow[0:1, c0:c0 + _S].T
        neg = jnp.where(neg < inf, neg, fbv)
        x = jnp.maximum(dist_q - (neg - _MARGIN), 0.0)
        posd = jnp.logical_and(eq_q[:, 0:_S], colv > rowv)
        sums.append(jnp.sum(jnp.where(posd, x[:, 0:_S], 0.0),
                            keepdims=True))
        if q + 1 < _NQ:
            sums.append(jnp.sum(jnp.where(eq_q[:, _S:], x[:, _S:], 0.0),
                                keepdims=True))

    total = sums[0]
    for s in sums[1:]:
        total = total + s
    sum_ref[...] = total / cnt.astype(jnp.float32)


def kernel(embeddings, target):
    labc = target.reshape(_B, 1)
    labr = target.reshape(1, _B)
    out_sum, out_cnt = pl.pallas_call(
        _triplet_kernel,
        out_shape=[
            jax.ShapeDtypeStruct((1, 1), jnp.float32),
            jax.ShapeDtypeStruct((1, 1), jnp.int32),
        ],
        scratch_shapes=[
            pltpu.VMEM((_B, _DA), jnp.float32),
            pltpu.VMEM((_B, _DA), jnp.float32),
        ],
    )(embeddings, labc, labr)
    return (out_sum[0, 0], out_cnt[0, 0])
